# Initial kernel scaffold; baseline (speedup 1.0000x reference)
import jax
import jax.numpy as jnp
from jax.experimental import pallas as pl


def kernel(x, embed):
    raise NotImplementedError("write your pallas kernel here")


# speedup vs baseline: 1.4410x; 1.4410x over previous
import functools

import jax
import jax.numpy as jnp
from jax.experimental import pallas as pl
from jax.experimental.pallas import tpu as pltpu
from jax.experimental.pallas import tpu_sc as plsc

_EMB = 256
_NCODE = 8192
_COMMIT = 0.25

_TM = 256
_GW = 128


def _dist_argmin_kernel(x_ref, e_ref, xn_ref, en_ref, idx_ref, msum_ref):
    i = pl.program_id(0)
    xblk = x_ref[...]
    emb = e_ref[...]
    xn = xn_ref[...]
    en = en_ref[...]

    mm = jnp.dot(xblk, emb, preferred_element_type=jnp.float32)
    dist = (xn - 2.0 * mm) + en

    m = jnp.min(dist, axis=1, keepdims=True)
    iota = jax.lax.broadcasted_iota(jnp.int32, dist.shape, 1)
    idx = jnp.min(jnp.where(dist == m, iota, _NCODE), axis=1)
    idx_ref[0, 0, :] = idx

    s = jnp.sum(m)

    @pl.when(i == 0)
    def _():
        msum_ref[0, 0] = s

    @pl.when(i != 0)
    def _():
        msum_ref[0, 0] += s


def _dist_argmin(flat, embed, xn, en):
    nt = flat.shape[0] // _TM
    return pl.pallas_call(
        _dist_argmin_kernel,
        grid=(nt,),
        in_specs=[
            pl.BlockSpec((_TM, _EMB), lambda i: (i, 0)),
            pl.BlockSpec((_EMB, _NCODE), lambda i: (0, 0)),
            pl.BlockSpec((_TM, 1), lambda i: (i, 0)),
            pl.BlockSpec((1, _NCODE), lambda i: (0, 0)),
        ],
        out_specs=[
            pl.BlockSpec((1, 1, _TM), lambda i: (i, 0, 0)),
            pl.BlockSpec((1, 1), lambda i: (0, 0),
                         memory_space=pltpu.SMEM),
        ],
        out_shape=[
            jax.ShapeDtypeStruct((nt, 1, _TM), jnp.int32),
            jax.ShapeDtypeStruct((1, 1), jnp.float32),
        ],
    )(flat, embed, xn, en)


def _gather_rows(embed_t, idx_row):
    ntok = idx_row.shape[1]
    mesh = plsc.VectorSubcoreMesh(core_axis_name="core",
                                  subcore_axis_name="subcore")

    @functools.partial(
        pl.kernel,
        out_type=jax.ShapeDtypeStruct((ntok, _EMB), jnp.float32),
        mesh=mesh,
    )
    def gk(e_hbm, i_hbm, o_hbm):
        def body(i_vmem, o_vmem):
            pltpu.sync_copy(e_hbm.at[i_vmem.at[0]], o_vmem)

        pltpu.emit_pipeline(
            body,
            grid=(ntok // _GW,),
            in_specs=[pl.BlockSpec((1, _GW), lambda i: (0, i))],
            out_specs=[pl.BlockSpec((_GW, _EMB), lambda i: (i, 0))],
            core_axis_name=("core", "subcore"),
            dimension_semantics=(pltpu.PARALLEL,),
        )(i_hbm, o_hbm)

    return gk(embed_t, idx_row)


def kernel(x, embed):
    B, C, H, W = x.shape
    ntok = B * H * W
    xp = jnp.transpose(x, (0, 2, 3, 1)).reshape(ntok, _EMB)
    xn = jnp.sum(xp**2, axis=1, keepdims=True)
    en = jnp.sum(embed**2, axis=0, keepdims=True)
    idx3, msum = _dist_argmin(xp, embed, xn, en)
    idx_flat = idx3.reshape(ntok)
    embed_idx = idx_flat.reshape(B, H, W)

    qflat = _gather_rows(embed.T, idx_flat.reshape(1, ntok))
    quantize = jnp.transpose(qflat.reshape(B, H, W, _EMB), (0, 3, 1, 2))

    m = msum[0, 0] / (ntok * _EMB)
    loss = m + _COMMIT * m
    return (quantize, loss, embed_idx)
